# Initial kernel scaffold; baseline (speedup 1.0000x reference)
import jax
import jax.numpy as jnp
from jax.experimental import pallas as pl


def kernel(x):
    raise NotImplementedError("write your pallas kernel here")


# speedup vs baseline: 14.4936x; 14.4936x over previous
import functools

import jax
import jax.numpy as jnp
from jax.experimental import pallas as pl
from jax.experimental.pallas import tpu as pltpu

_B, _C, _N = 2, 256, 8192
_K = 9
_DIL = 2
_KK = _K * _DIL
_R = 128


def _norm_body(x_ref, xn_ref, sq_ref):
    x = x_ref[0]
    n = jnp.sqrt(jnp.sum(x * x, axis=0, keepdims=True))
    xn = x / jnp.maximum(n, 1e-12)
    xn_ref[0] = xn
    sq_ref[0] = jnp.sum(xn * xn, axis=0, keepdims=True)


def _topk_body(xq_ref, xk_ref, sqr_ref, sqa_ref, out_ref):
    xq = xq_ref[0]
    xk = xk_ref[0]
    inner = -2.0 * jax.lax.dot_general(
        xq, xk, (((1,), (0,)), ((), ())),
        preferred_element_type=jnp.float32)
    dist = (sqr_ref[0] + inner) + sqa_ref[0]
    v = -dist
    iota = jax.lax.broadcasted_iota(jnp.int32, (_R, _N), 1)
    for j in range(_KK):
        m = jnp.max(v, axis=1, keepdims=True)
        am = jnp.min(jnp.where(v == m, iota, _N), axis=1, keepdims=True)
        if j % _DIL == 0:
            out_ref[0, :, j // _DIL : j // _DIL + 1] = am
        v = jnp.where(iota == am, -jnp.inf, v)


@jax.jit
def kernel(x):
    xr = jnp.reshape(x, (_B, _C, _N))
    xn, sq = pl.pallas_call(
        _norm_body,
        grid=(_B,),
        in_specs=[pl.BlockSpec((1, _C, _N), lambda b: (b, 0, 0))],
        out_specs=[
            pl.BlockSpec((1, _C, _N), lambda b: (b, 0, 0)),
            pl.BlockSpec((1, 1, _N), lambda b: (b, 0, 0)),
        ],
        out_shape=[
            jax.ShapeDtypeStruct((_B, _C, _N), jnp.float32),
            jax.ShapeDtypeStruct((_B, 1, _N), jnp.float32),
        ],
    )(xr)

    xn_nc = jnp.swapaxes(xn, 1, 2)
    sq_n1 = jnp.swapaxes(sq, 1, 2)

    nn_idx = pl.pallas_call(
        _topk_body,
        grid=(_B, _N // _R),
        in_specs=[
            pl.BlockSpec((1, _R, _C), lambda b, i: (b, i, 0)),
            pl.BlockSpec((1, _C, _N), lambda b, i: (b, 0, 0)),
            pl.BlockSpec((1, _R, 1), lambda b, i: (b, i, 0)),
            pl.BlockSpec((1, 1, _N), lambda b, i: (b, 0, 0)),
        ],
        out_specs=pl.BlockSpec((1, _R, _K), lambda b, i: (b, i, 0)),
        out_shape=jax.ShapeDtypeStruct((_B, _N, _K), jnp.int32),
    )(xn_nc, xn, sq_n1, sq)

    center = jnp.broadcast_to(
        jnp.arange(_N, dtype=jnp.int32)[None, :, None], (_B, _N, _K))
    return jnp.stack((nn_idx, center), axis=0)


# speedup vs baseline: 20.7789x; 1.4337x over previous
import functools

import jax
import jax.numpy as jnp
from jax.experimental import pallas as pl
from jax.experimental.pallas import tpu as pltpu

_B, _C, _N = 2, 256, 8192
_K = 9
_DIL = 2
_KK = _K * _DIL
_R = 128
_D = 16
_NCH = _N // _D
_NEG = -1e30


def _norm_body(x_ref, xn_ref, sq_ref):
    x = x_ref[0]
    n = jnp.sqrt(jnp.sum(x * x, axis=0, keepdims=True))
    xn = x / jnp.maximum(n, 1e-12)
    xn_ref[0] = xn
    sq_ref[0] = jnp.sum(xn * xn, axis=0, keepdims=True)


def _topk_body(xq_ref, xk_ref, sqr_ref, sqa_ref, out_ref):
    xq = xq_ref[0]
    xk = xk_ref[0]
    inner = -2.0 * jax.lax.dot_general(
        xq, xk, (((1,), (0,)), ((), ())),
        preferred_element_type=jnp.float32)
    dist = (sqr_ref[0] + inner) + sqa_ref[0]
    v = -dist

    vs = [v[:, a * _NCH:(a + 1) * _NCH] for a in range(_D)]
    ms, args = [], []
    for _t in range(3):
        m = vs[0]
        arg = jnp.zeros((_R, _NCH), jnp.int32)
        for a in range(1, _D):
            gt = vs[a] > m
            m = jnp.where(gt, vs[a], m)
            arg = jnp.where(gt, a, arg)
        for a in range(_D):
            vs[a] = jnp.where(arg == a, _NEG, vs[a])
        ms.append(m)
        args.append(arg)

    lane = jax.lax.broadcasted_iota(jnp.int32, (_R, _NCH), 1)
    m1, m2, m3 = ms
    g1 = args[0] * _NCH + lane
    g2 = args[1] * _NCH + lane
    g3 = args[2] * _NCH + lane
    bound3 = m3
    fb = jnp.zeros((_R, _NCH), jnp.bool_)
    bigi = jnp.int32(1 << 30)
    for j in range(_KK):
        w = jnp.max(m1, axis=1, keepdims=True)
        fb = fb | ((m1 == _NEG) & (bound3 >= w))
        wg = jnp.min(jnp.where(m1 == w, g1, bigi), axis=1, keepdims=True)
        if j % _DIL == 0:
            out_ref[0, :, j // _DIL:j // _DIL + 1] = wg
        hit = g1 == wg
        m1 = jnp.where(hit, m2, m1)
        m2 = jnp.where(hit, m3, m2)
        m3 = jnp.where(hit, _NEG, m3)
        g1 = jnp.where(hit, g2, g1)
        g2 = jnp.where(hit, g3, g2)

    @pl.when(jnp.any(fb))
    def _fallback():
        vv = v
        iota = jax.lax.broadcasted_iota(jnp.int32, (_R, _N), 1)
        for j in range(_KK):
            m = jnp.max(vv, axis=1, keepdims=True)
            am = jnp.min(jnp.where(vv == m, iota, _N), axis=1, keepdims=True)
            if j % _DIL == 0:
                out_ref[0, :, j // _DIL:j // _DIL + 1] = am
            vv = jnp.where(iota == am, _NEG, vv)


@jax.jit
def kernel(x):
    xr = jnp.reshape(x, (_B, _C, _N))
    xn, sq = pl.pallas_call(
        _norm_body,
        grid=(_B,),
        in_specs=[pl.BlockSpec((1, _C, _N), lambda b: (b, 0, 0))],
        out_specs=[
            pl.BlockSpec((1, _C, _N), lambda b: (b, 0, 0)),
            pl.BlockSpec((1, 1, _N), lambda b: (b, 0, 0)),
        ],
        out_shape=[
            jax.ShapeDtypeStruct((_B, _C, _N), jnp.float32),
            jax.ShapeDtypeStruct((_B, 1, _N), jnp.float32),
        ],
    )(xr)

    xk_bf = xn.astype(jnp.bfloat16)
    xq_bf = jnp.swapaxes(xn, 1, 2).astype(jnp.bfloat16)
    sq_n1 = jnp.swapaxes(sq, 1, 2)

    nn_idx = pl.pallas_call(
        _topk_body,
        grid=(_B, _N // _R),
        in_specs=[
            pl.BlockSpec((1, _R, _C), lambda b, i: (b, i, 0)),
            pl.BlockSpec((1, _C, _N), lambda b, i: (b, 0, 0)),
            pl.BlockSpec((1, _R, 1), lambda b, i: (b, i, 0)),
            pl.BlockSpec((1, 1, _N), lambda b, i: (b, 0, 0)),
        ],
        out_specs=pl.BlockSpec((1, _R, _K), lambda b, i: (b, i, 0)),
        out_shape=jax.ShapeDtypeStruct((_B, _N, _K), jnp.int32),
    )(xq_bf, xk_bf, sq_n1, sq)

    center = jnp.broadcast_to(
        jnp.arange(_N, dtype=jnp.int32)[None, :, None], (_B, _N, _K))
    return jnp.stack((nn_idx, center), axis=0)


# speedup vs baseline: 27.6969x; 1.3329x over previous
import functools

import jax
import jax.numpy as jnp
from jax.experimental import pallas as pl
from jax.experimental.pallas import tpu as pltpu

_B, _C, _N = 2, 256, 8192
_K = 9
_DIL = 2
_KK = _K * _DIL
_R = 128
_NCH = 128
_D = _N // _NCH
_T = 4
_POS = 1e30


def _norm_body(x_ref, xn_ref, sq_ref):
    x = x_ref[0]
    n = jnp.sqrt(jnp.sum(x * x, axis=0, keepdims=True))
    xn = x / jnp.maximum(n, 1e-12)
    xn_ref[0] = xn
    sq_ref[0] = jnp.sum(xn * xn, axis=0, keepdims=True)


def _topk_body(xq_ref, xk_ref, sqr_ref, sqa_ref, out_ref):
    xq = xq_ref[0]
    xk = xk_ref[0]
    inner = -2.0 * jax.lax.dot_general(
        xq, xk, (((1,), (0,)), ((), ())),
        preferred_element_type=jnp.float32)
    dist = (sqr_ref[0] + inner) + sqa_ref[0]

    ms, args = [], []
    for t in range(_T):
        excl0 = None
        for p in range(t):
            e = args[p] == 0.0
            excl0 = e if excl0 is None else (excl0 | e)
        s0 = dist[:, 0:_NCH]
        m = s0 if excl0 is None else jnp.where(excl0, _POS, s0)
        arg = jnp.zeros((_R, _NCH), jnp.float32)
        for a in range(1, _D):
            s = dist[:, a * _NCH:(a + 1) * _NCH]
            ok = s < m
            for p in range(t):
                ok = ok & (args[p] != float(a))
            m = jnp.where(ok, s, m)
            arg = jnp.where(ok, float(a), arg)
        ms.append(m)
        args.append(arg)

    lane = jax.lax.broadcasted_iota(
        jnp.int32, (_R, _NCH), 1).astype(jnp.float32)
    m1, m2, m3, m4 = ms
    g1 = args[0] * float(_NCH) + lane
    g2 = args[1] * float(_NCH) + lane
    g3 = args[2] * float(_NCH) + lane
    g4 = args[3] * float(_NCH) + lane
    bound4 = m4
    bigg = 3.0e7
    w = None
    for j in range(_KK):
        w = jnp.min(m1, axis=1, keepdims=True)
        wg = jnp.min(jnp.where(m1 == w, g1, bigg), axis=1, keepdims=True)
        if j % _DIL == 0:
            out_ref[0, :, j // _DIL:j // _DIL + 1] = wg.astype(jnp.int32)
        hit = g1 == wg
        m1 = jnp.where(hit, m2, m1)
        m2 = jnp.where(hit, m3, m2)
        m3 = jnp.where(hit, m4, m3)
        m4 = jnp.where(hit, _POS, m4)
        g1 = jnp.where(hit, g2, g1)
        g2 = jnp.where(hit, g3, g2)
        g3 = jnp.where(hit, g4, g3)

    fb = jnp.any((m1 == _POS) & (bound4 <= w))

    @pl.when(fb)
    def _fallback():
        vv = dist
        iota = jax.lax.broadcasted_iota(
            jnp.int32, (_R, _N), 1).astype(jnp.float32)
        for j in range(_KK):
            m = jnp.min(vv, axis=1, keepdims=True)
            am = jnp.min(jnp.where(vv == m, iota, 3.0e7), axis=1,
                         keepdims=True)
            if j % _DIL == 0:
                out_ref[0, :, j // _DIL:j // _DIL + 1] = am.astype(jnp.int32)
            vv = jnp.where(iota == am, _POS, vv)


@jax.jit
def kernel(x):
    xr = jnp.reshape(x, (_B, _C, _N))
    xn, sq = pl.pallas_call(
        _norm_body,
        grid=(_B,),
        in_specs=[pl.BlockSpec((1, _C, _N), lambda b: (b, 0, 0))],
        out_specs=[
            pl.BlockSpec((1, _C, _N), lambda b: (b, 0, 0)),
            pl.BlockSpec((1, 1, _N), lambda b: (b, 0, 0)),
        ],
        out_shape=[
            jax.ShapeDtypeStruct((_B, _C, _N), jnp.float32),
            jax.ShapeDtypeStruct((_B, 1, _N), jnp.float32),
        ],
    )(xr)

    xk_bf = xn.astype(jnp.bfloat16)
    xq_bf = jnp.swapaxes(xn, 1, 2).astype(jnp.bfloat16)
    sq_n1 = jnp.swapaxes(sq, 1, 2)

    nn_idx = pl.pallas_call(
        _topk_body,
        grid=(_B, _N // _R),
        in_specs=[
            pl.BlockSpec((1, _R, _C), lambda b, i: (b, i, 0)),
            pl.BlockSpec((1, _C, _N), lambda b, i: (b, 0, 0)),
            pl.BlockSpec((1, _R, 1), lambda b, i: (b, i, 0)),
            pl.BlockSpec((1, 1, _N), lambda b, i: (b, 0, 0)),
        ],
        out_specs=pl.BlockSpec((1, _R, _K), lambda b, i: (b, i, 0)),
        out_shape=jax.ShapeDtypeStruct((_B, _N, _K), jnp.int32),
    )(xq_bf, xk_bf, sq_n1, sq)

    center = jnp.broadcast_to(
        jnp.arange(_N, dtype=jnp.int32)[None, :, None], (_B, _N, _K))
    return jnp.stack((nn_idx, center), axis=0)
